# Initial kernel scaffold; baseline (speedup 1.0000x reference)
import jax
import jax.numpy as jnp
from jax.experimental import pallas as pl


def kernel(x, edge_index, W, bias):
    raise NotImplementedError("write your pallas kernel here")


# speedup vs baseline: 23.9220x; 23.9220x over previous
import functools

import jax
import jax.numpy as jnp
from jax import lax
from jax.experimental import pallas as pl
from jax.experimental.pallas import tpu as pltpu
from jax.experimental.pallas import tpu_sc as plsc

NC = 2
NS = 16
NW = NC * NS
CHUNK = 80


def _sc_mesh():
    return plsc.VectorSubcoreMesh(core_axis_name="c", subcore_axis_name="s")


def _hist(col3d, npad):
    _, chunks_per_w, chunk = col3d.shape
    rp = npad // NS

    @functools.partial(
        pl.kernel,
        out_type=jax.ShapeDtypeStruct((NC * npad,), jnp.float32),
        mesh=_sc_mesh(),
        compiler_params=pltpu.CompilerParams(needs_layout_passes=False),
        scratch_types=[
            pltpu.VMEM((chunks_per_w, chunk), jnp.int32),
            pltpu.VMEM((npad,), jnp.float32),
            pltpu.VMEM((rp,), jnp.float32),
            pltpu.VMEM_SHARED((NS * npad,), jnp.float32),
        ],
    )
    def hist_kernel(col_hbm, deg_hbm, idx_v, cnt_v, tmp_v, stage_sh):
        c = lax.axis_index("c")
        s = lax.axis_index("s")
        wid = s * NC + c
        pltpu.sync_copy(col_hbm.at[wid], idx_v)

        @pl.loop(0, npad // 16)
        def _(i):
            cnt_v[pl.ds(i * 16, 16)] = jnp.zeros((16,), jnp.float32)

        @pl.loop(0, chunks_per_w)
        def _(j):
            @pl.loop(0, chunk // 16)
            def _(k):
                idx = idx_v[j, pl.ds(k * 16, 16)]
                plsc.addupdate_scatter(cnt_v, [idx],
                                       jnp.full((16,), 1.0, jnp.float32))

        pltpu.sync_copy(cnt_v, stage_sh.at[pl.ds(s * npad, npad)])
        plsc.subcore_barrier()
        pltpu.sync_copy(stage_sh.at[pl.ds(s * rp, rp)], cnt_v.at[pl.ds(0, rp)])

        @pl.loop(1, NS)
        def _(t):
            pltpu.sync_copy(stage_sh.at[pl.ds(t * npad + s * rp, rp)], tmp_v)

            @pl.loop(0, rp // 16)
            def _(i):
                cnt_v[pl.ds(i * 16, 16)] = (cnt_v[pl.ds(i * 16, 16)]
                                            + tmp_v[pl.ds(i * 16, 16)])

        pltpu.sync_copy(cnt_v.at[pl.ds(0, rp)],
                        deg_hbm.at[pl.ds(c * npad + s * rp, rp)])

    return hist_kernel(col3d)


def _agg(xws, row3d, col3d, zeros):
    cdim = xws.shape[1]
    _, chunks_per_w, chunk = col3d.shape
    npad = zeros.shape[0]
    rp = npad // NS

    @functools.partial(
        pl.kernel,
        out_type=jax.ShapeDtypeStruct((NC, npad, cdim), jnp.float32),
        mesh=_sc_mesh(),
        scratch_types=[
            pltpu.VMEM((chunks_per_w, chunk), jnp.int32),
            pltpu.VMEM((chunks_per_w, chunk), jnp.int32),
            pltpu.VMEM((chunk, cdim), jnp.float32),
            pltpu.VMEM_SHARED((npad, cdim), jnp.float32),
            pltpu.SemaphoreType.DMA,
        ],
    )
    def agg_kernel(xws_hbm, row_hbm, col_hbm, z_hbm, out_hbm,
                   row_v, col_v, rows_v, acc_sh, sem):
        c = lax.axis_index("c")
        s = lax.axis_index("s")
        wid = s * NC + c
        pltpu.sync_copy(z_hbm.at[pl.ds(s * rp, rp)], acc_sh.at[pl.ds(s * rp, rp)])
        pltpu.sync_copy(row_hbm.at[wid], row_v)
        pltpu.sync_copy(col_hbm.at[wid], col_v)
        plsc.subcore_barrier()

        @pl.loop(0, chunks_per_w)
        def _(j):
            pltpu.async_copy(xws_hbm.at[col_v.at[j]], rows_v, sem).wait()
            pltpu.sync_copy(rows_v, acc_sh.at[row_v.at[j]], add=True)

        plsc.subcore_barrier()
        pltpu.sync_copy(acc_sh.at[pl.ds(s * rp, rp)],
                        out_hbm.at[c, pl.ds(s * rp, rp)])

    return agg_kernel(xws, row3d, col3d, zeros)


def _matmul(x, w):
    n, cin = x.shape
    cout = w.shape[0]
    blk = 1000

    def body(x_ref, w_ref, o_ref):
        o_ref[...] = lax.dot_general(
            x_ref[...], w_ref[...], (((1,), (1,)), ((), ())),
            preferred_element_type=jnp.float32)

    return pl.pallas_call(
        body,
        grid=(n // blk,),
        in_specs=[pl.BlockSpec((blk, cin), lambda i: (i, 0)),
                  pl.BlockSpec((cout, cin), lambda i: (0, 0))],
        out_specs=pl.BlockSpec((blk, cout), lambda i: (i, 0)),
        out_shape=jax.ShapeDtypeStruct((n, cout), jnp.float32),
    )(x, w)


def _dis_from(d0, d1):
    deg = d0 + d1
    return jnp.where(deg > 0.0, lax.rsqrt(deg), 0.0)


def _scale(xw, d0, d1):
    n, cdim = xw.shape
    blk = 1000

    def body(xw_ref, d0_ref, d1_ref, o_ref):
        o_ref[...] = xw_ref[...] * _dis_from(d0_ref[...], d1_ref[...])

    return pl.pallas_call(
        body,
        grid=(n // blk,),
        in_specs=[pl.BlockSpec((blk, cdim), lambda i: (i, 0)),
                  pl.BlockSpec((blk, 1), lambda i: (i, 0)),
                  pl.BlockSpec((blk, 1), lambda i: (i, 0))],
        out_specs=pl.BlockSpec((blk, cdim), lambda i: (i, 0)),
        out_shape=jax.ShapeDtypeStruct((n, cdim), jnp.float32),
    )(xw, d0, d1)


def _finalize(parts, d0, d1, bias2d, n):
    cdim = parts.shape[2]
    blk = 1000

    def body(p_ref, d0_ref, d1_ref, b_ref, o_ref):
        p = p_ref[0] + p_ref[1]
        o_ref[...] = p * _dis_from(d0_ref[...], d1_ref[...]) + b_ref[...]

    return pl.pallas_call(
        body,
        grid=(n // blk,),
        in_specs=[pl.BlockSpec((NC, blk, cdim), lambda i: (0, i, 0)),
                  pl.BlockSpec((blk, 1), lambda i: (i, 0)),
                  pl.BlockSpec((blk, 1), lambda i: (i, 0)),
                  pl.BlockSpec((1, cdim), lambda i: (0, 0))],
        out_specs=pl.BlockSpec((blk, cdim), lambda i: (i, 0)),
        out_shape=jax.ShapeDtypeStruct((n, cdim), jnp.float32),
    )(parts, d0, d1, bias2d)


def kernel(x, edge_index, W, bias):
    n, cdim = x.shape
    e = edge_index.shape[1]
    npad = ((n + 16 * NS - 1) // (16 * NS)) * (16 * NS)
    row3d = edge_index[0].reshape(NW, e // (NW * CHUNK), CHUNK)
    col3d = edge_index[1].reshape(NW, e // (NW * CHUNK), CHUNK)
    zeros = jnp.zeros((npad, cdim), jnp.float32)

    deg_flat = _hist(col3d, npad)
    d0 = deg_flat[:npad].reshape(npad, 1)
    d1 = deg_flat[npad:].reshape(npad, 1)
    xw = _matmul(x, W)
    xws = _scale(xw, d0, d1)
    parts = _agg(xws, row3d, col3d, zeros)
    return _finalize(parts, d0, d1, bias.reshape(1, cdim), n)


# speedup vs baseline: 29.4897x; 1.2327x over previous
import functools

import jax
import jax.numpy as jnp
from jax import lax
from jax.experimental import pallas as pl
from jax.experimental.pallas import tpu as pltpu
from jax.experimental.pallas import tpu_sc as plsc

NC = 2
NS = 16
NW = NC * NS
CHUNK = 80


def _sc_mesh():
    return plsc.VectorSubcoreMesh(core_axis_name="c", subcore_axis_name="s")


def _hist(col2d, npad):
    _, _, per_w = col2d.shape
    rp = npad // NS

    @functools.partial(
        pl.kernel,
        out_type=jax.ShapeDtypeStruct((NC * npad,), jnp.float32),
        mesh=_sc_mesh(),
        compiler_params=pltpu.CompilerParams(needs_layout_passes=False),
        scratch_types=[
            pltpu.VMEM((1, per_w), jnp.int32),
            pltpu.VMEM((npad,), jnp.float32),
            pltpu.VMEM((rp,), jnp.float32),
            pltpu.VMEM_SHARED((NS * npad,), jnp.float32),
        ],
    )
    def hist_kernel(col_hbm, deg_hbm, idx_v, cnt_v, tmp_v, stage_sh):
        c = lax.axis_index("c")
        s = lax.axis_index("s")
        wid = s * NC + c
        pltpu.sync_copy(col_hbm.at[wid], idx_v)

        @pl.loop(0, npad // 16)
        def _(i):
            cnt_v[pl.ds(i * 16, 16)] = jnp.zeros((16,), jnp.float32)

        @pl.loop(0, per_w // 16)
        def _(k):
            idx = idx_v[0, pl.ds(k * 16, 16)]
            plsc.addupdate_scatter(cnt_v, [idx],
                                   jnp.full((16,), 1.0, jnp.float32))

        pltpu.sync_copy(cnt_v, stage_sh.at[pl.ds(s * npad, npad)])
        plsc.subcore_barrier()
        pltpu.sync_copy(stage_sh.at[pl.ds(s * rp, rp)], cnt_v.at[pl.ds(0, rp)])

        @pl.loop(1, NS)
        def _(t):
            pltpu.sync_copy(stage_sh.at[pl.ds(t * npad + s * rp, rp)], tmp_v)

            @pl.loop(0, rp // 16)
            def _(i):
                cnt_v[pl.ds(i * 16, 16)] = (cnt_v[pl.ds(i * 16, 16)]
                                            + tmp_v[pl.ds(i * 16, 16)])

        pltpu.sync_copy(cnt_v.at[pl.ds(0, rp)],
                        deg_hbm.at[pl.ds(c * npad + s * rp, rp)])

    return hist_kernel(col2d)


def _agg(xws, row3d, col1d, zeros):
    cdim = xws.shape[1]
    _, chunks_per_w, chunk = row3d.shape
    per_w = chunks_per_w * chunk
    npad = zeros.shape[0]
    rp = npad // NS

    assert chunks_per_w % 2 == 1
    npairs = chunks_per_w // 2

    @functools.partial(
        pl.kernel,
        out_type=jax.ShapeDtypeStruct((NC, npad, cdim), jnp.float32),
        mesh=_sc_mesh(),
        scratch_types=[
            pltpu.VMEM((chunks_per_w, chunk), jnp.int32),
            pltpu.VMEM((per_w,), jnp.int32),
            pltpu.VMEM((chunk, cdim), jnp.float32),
            pltpu.VMEM((chunk, cdim), jnp.float32),
            pltpu.VMEM_SHARED((npad, cdim), jnp.float32),
            pltpu.SemaphoreType.DMA,
            pltpu.SemaphoreType.DMA,
            pltpu.SemaphoreType.DMA,
            pltpu.SemaphoreType.DMA,
        ],
    )
    def agg_kernel(xws_hbm, row_hbm, col_hbm, z_hbm, out_hbm,
                   row_v, col_v, buf_a, buf_b, acc_sh, ga, gb, sa, sb):
        c = lax.axis_index("c")
        s = lax.axis_index("s")
        wid = s * NC + c
        pltpu.sync_copy(row_hbm.at[wid], row_v)
        pltpu.sync_copy(col_hbm.at[pl.ds(wid * per_w, per_w)], col_v)

        def gather(j, buf, sem):
            pltpu.async_copy(xws_hbm.at[col_v.at[pl.ds(j * chunk, chunk)]],
                             buf, sem)

        def wait(buf, sem):
            pltpu.make_async_copy(xws_hbm.at[col_v.at[pl.ds(0, chunk)]],
                                  buf, sem).wait()

        gather(0, buf_a, ga)
        gather(1, buf_b, gb)
        pltpu.sync_copy(z_hbm.at[pl.ds(s * rp, rp)], acc_sh.at[pl.ds(s * rp, rp)])
        plsc.subcore_barrier()

        @pl.loop(0, npairs)
        def _(p):
            j0 = 2 * p
            wait(buf_a, ga)
            pltpu.async_copy(buf_a, acc_sh.at[row_v.at[j0]], sa, add=True)
            wait(buf_b, gb)
            pltpu.async_copy(buf_b, acc_sh.at[row_v.at[j0 + 1]], sb, add=True)
            wait(buf_a, sa)
            gather(j0 + 2, buf_a, ga)
            wait(buf_b, sb)

            @pl.when(j0 + 3 < chunks_per_w)
            def _():
                gather(j0 + 3, buf_b, gb)

        wait(buf_a, ga)
        pltpu.sync_copy(buf_a, acc_sh.at[row_v.at[chunks_per_w - 1]], add=True)
        plsc.subcore_barrier()
        pltpu.sync_copy(acc_sh.at[pl.ds(s * rp, rp)],
                        out_hbm.at[c, pl.ds(s * rp, rp)])

    return agg_kernel(xws, row3d, col1d, zeros)


def _matmul(x, w):
    n, cin = x.shape
    cout = w.shape[0]
    blk = 1000

    def body(x_ref, w_ref, o_ref):
        o_ref[...] = lax.dot_general(
            x_ref[...], w_ref[...], (((1,), (1,)), ((), ())),
            preferred_element_type=jnp.float32)

    return pl.pallas_call(
        body,
        grid=(n // blk,),
        in_specs=[pl.BlockSpec((blk, cin), lambda i: (i, 0)),
                  pl.BlockSpec((cout, cin), lambda i: (0, 0))],
        out_specs=pl.BlockSpec((blk, cout), lambda i: (i, 0)),
        out_shape=jax.ShapeDtypeStruct((n, cout), jnp.float32),
    )(x, w)


def _dis_from(d0, d1):
    deg = d0 + d1
    return jnp.where(deg > 0.0, lax.rsqrt(deg), 0.0)


def _scale(xw, d0, d1):
    n, cdim = xw.shape
    blk = 1000

    def body(xw_ref, d0_ref, d1_ref, o_ref):
        o_ref[...] = xw_ref[...] * _dis_from(d0_ref[...], d1_ref[...])

    return pl.pallas_call(
        body,
        grid=(n // blk,),
        in_specs=[pl.BlockSpec((blk, cdim), lambda i: (i, 0)),
                  pl.BlockSpec((blk, 1), lambda i: (i, 0)),
                  pl.BlockSpec((blk, 1), lambda i: (i, 0))],
        out_specs=pl.BlockSpec((blk, cdim), lambda i: (i, 0)),
        out_shape=jax.ShapeDtypeStruct((n, cdim), jnp.float32),
    )(xw, d0, d1)


def _finalize(parts, d0, d1, bias2d, n):
    cdim = parts.shape[2]
    blk = 1000

    def body(p_ref, d0_ref, d1_ref, b_ref, o_ref):
        p = p_ref[0] + p_ref[1]
        o_ref[...] = p * _dis_from(d0_ref[...], d1_ref[...]) + b_ref[...]

    return pl.pallas_call(
        body,
        grid=(n // blk,),
        in_specs=[pl.BlockSpec((NC, blk, cdim), lambda i: (0, i, 0)),
                  pl.BlockSpec((blk, 1), lambda i: (i, 0)),
                  pl.BlockSpec((blk, 1), lambda i: (i, 0)),
                  pl.BlockSpec((1, cdim), lambda i: (0, 0))],
        out_specs=pl.BlockSpec((blk, cdim), lambda i: (i, 0)),
        out_shape=jax.ShapeDtypeStruct((n, cdim), jnp.float32),
    )(parts, d0, d1, bias2d)


def kernel(x, edge_index, W, bias):
    n, cdim = x.shape
    e = edge_index.shape[1]
    npad = ((n + 16 * NS - 1) // (16 * NS)) * (16 * NS)
    row3d = edge_index[0].reshape(NW, e // (NW * CHUNK), CHUNK)
    col1d = edge_index[1]
    zeros = jnp.zeros((npad, cdim), jnp.float32)

    deg_flat = _hist(edge_index[1].reshape(NW, 1, e // NW), npad)
    d0 = deg_flat[:npad].reshape(npad, 1)
    d1 = deg_flat[npad:].reshape(npad, 1)
    xw = _matmul(x, W)
    xws = _scale(xw, d0, d1)
    parts = _agg(xws, row3d, col1d, zeros)
    return _finalize(parts, d0, d1, bias.reshape(1, cdim), n)


# speedup vs baseline: 35.7362x; 1.2118x over previous
import functools

import jax
import jax.numpy as jnp
from jax import lax
from jax.experimental import pallas as pl
from jax.experimental.pallas import tpu as pltpu
from jax.experimental.pallas import tpu_sc as plsc

NC = 2
NS = 16
NW = NC * NS
CHUNK = 40
NBUF = 5


def _sc_mesh():
    return plsc.VectorSubcoreMesh(core_axis_name="c", subcore_axis_name="s")


def _hist(col2d, npad):
    _, _, per_w = col2d.shape
    rp = npad // NS

    @functools.partial(
        pl.kernel,
        out_type=jax.ShapeDtypeStruct((NC * npad,), jnp.float32),
        mesh=_sc_mesh(),
        compiler_params=pltpu.CompilerParams(needs_layout_passes=False),
        scratch_types=[
            pltpu.VMEM((1, per_w), jnp.int32),
            pltpu.VMEM((npad,), jnp.float32),
            pltpu.VMEM((rp,), jnp.float32),
            pltpu.VMEM_SHARED((NS * npad,), jnp.float32),
        ],
    )
    def hist_kernel(col_hbm, deg_hbm, idx_v, cnt_v, tmp_v, stage_sh):
        c = lax.axis_index("c")
        s = lax.axis_index("s")
        wid = s * NC + c
        pltpu.sync_copy(col_hbm.at[wid], idx_v)

        @pl.loop(0, npad // 16)
        def _(i):
            cnt_v[pl.ds(i * 16, 16)] = jnp.zeros((16,), jnp.float32)

        @pl.loop(0, per_w // 16)
        def _(k):
            idx = idx_v[0, pl.ds(k * 16, 16)]
            plsc.addupdate_scatter(cnt_v, [idx],
                                   jnp.full((16,), 1.0, jnp.float32))

        pltpu.sync_copy(cnt_v, stage_sh.at[pl.ds(s * npad, npad)])
        plsc.subcore_barrier()
        pltpu.sync_copy(stage_sh.at[pl.ds(s * rp, rp)], cnt_v.at[pl.ds(0, rp)])

        @pl.loop(1, NS)
        def _(t):
            pltpu.sync_copy(stage_sh.at[pl.ds(t * npad + s * rp, rp)], tmp_v)

            @pl.loop(0, rp // 16)
            def _(i):
                cnt_v[pl.ds(i * 16, 16)] = (cnt_v[pl.ds(i * 16, 16)]
                                            + tmp_v[pl.ds(i * 16, 16)])

        pltpu.sync_copy(cnt_v.at[pl.ds(0, rp)],
                        deg_hbm.at[pl.ds(c * npad + s * rp, rp)])

    return hist_kernel(col2d)


def _agg(xws, row1d, col1d, zeros):
    cdim = xws.shape[1]
    chunk = CHUNK
    per_w = row1d.shape[0] // NW
    chunks_per_w = per_w // chunk
    npad = zeros.shape[0]
    rp = npad // NS

    assert chunks_per_w % NBUF == 0
    nsteps = chunks_per_w // NBUF

    @functools.partial(
        pl.kernel,
        out_type=jax.ShapeDtypeStruct((NC, npad, cdim), jnp.float32),
        mesh=_sc_mesh(),
        scratch_types=[
            pltpu.VMEM((per_w,), jnp.int32),
            pltpu.VMEM((per_w,), jnp.int32),
        ] + [pltpu.VMEM((chunk, cdim), jnp.float32)] * NBUF
          + [pltpu.VMEM_SHARED((npad, cdim), jnp.float32)]
          + [pltpu.SemaphoreType.DMA] * (2 * NBUF),
    )
    def agg_kernel(xws_hbm, row_hbm, col_hbm, z_hbm, out_hbm,
                   row_v, col_v, *rest):
        bufs = rest[:NBUF]
        acc_sh = rest[NBUF]
        gsem = rest[NBUF + 1:2 * NBUF + 1]
        ssem = rest[2 * NBUF + 1:]
        c = lax.axis_index("c")
        s = lax.axis_index("s")
        wid = s * NC + c
        pltpu.sync_copy(row_hbm.at[pl.ds(wid * per_w, per_w)], row_v)
        pltpu.sync_copy(col_hbm.at[pl.ds(wid * per_w, per_w)], col_v)

        def gather(j, b):
            pltpu.async_copy(xws_hbm.at[col_v.at[pl.ds(j * chunk, chunk)]],
                             bufs[b], gsem[b])

        def scatter(j, b):
            pltpu.async_copy(bufs[b],
                             acc_sh.at[row_v.at[pl.ds(j * chunk, chunk)]],
                             ssem[b], add=True)

        def wait(b, sem):
            pltpu.make_async_copy(xws_hbm.at[col_v.at[pl.ds(0, chunk)]],
                                  bufs[b], sem).wait()

        for b in range(NBUF):
            gather(b, b)
        pltpu.sync_copy(z_hbm.at[pl.ds(s * rp, rp)], acc_sh.at[pl.ds(s * rp, rp)])
        plsc.subcore_barrier()

        @pl.loop(0, nsteps)
        def _(p):
            j0 = p * NBUF
            for b in range(NBUF):
                wait(b, gsem[b])
                scatter(j0 + b, b)
            for b in range(NBUF):
                wait(b, ssem[b])
                jn = j0 + NBUF + b

                @pl.when(jn < chunks_per_w)
                def _():
                    gather(jn, b)

        plsc.subcore_barrier()
        pltpu.sync_copy(acc_sh.at[pl.ds(s * rp, rp)],
                        out_hbm.at[c, pl.ds(s * rp, rp)])

    return agg_kernel(xws, row1d, col1d, zeros)


def _dis_from(d0, d1):
    deg = d0 + d1
    return jnp.where(deg > 0.0, lax.rsqrt(deg), 0.0)


def _matmul_scale(x, w, d0, d1):
    n, cin = x.shape
    cout = w.shape[0]
    blk = 1000

    def body(x_ref, w_ref, d0_ref, d1_ref, o_ref):
        xw = lax.dot_general(
            x_ref[...], w_ref[...], (((1,), (1,)), ((), ())),
            preferred_element_type=jnp.float32)
        o_ref[...] = xw * _dis_from(d0_ref[...], d1_ref[...])

    return pl.pallas_call(
        body,
        grid=(n // blk,),
        in_specs=[pl.BlockSpec((blk, cin), lambda i: (i, 0)),
                  pl.BlockSpec((cout, cin), lambda i: (0, 0)),
                  pl.BlockSpec((blk, 1), lambda i: (i, 0)),
                  pl.BlockSpec((blk, 1), lambda i: (i, 0))],
        out_specs=pl.BlockSpec((blk, cout), lambda i: (i, 0)),
        out_shape=jax.ShapeDtypeStruct((n, cout), jnp.float32),
    )(x, w, d0, d1)


def _finalize(parts, d0, d1, bias2d, n):
    cdim = parts.shape[2]
    blk = 1000

    def body(p_ref, d0_ref, d1_ref, b_ref, o_ref):
        p = p_ref[0] + p_ref[1]
        o_ref[...] = p * _dis_from(d0_ref[...], d1_ref[...]) + b_ref[...]

    return pl.pallas_call(
        body,
        grid=(n // blk,),
        in_specs=[pl.BlockSpec((NC, blk, cdim), lambda i: (0, i, 0)),
                  pl.BlockSpec((blk, 1), lambda i: (i, 0)),
                  pl.BlockSpec((blk, 1), lambda i: (i, 0)),
                  pl.BlockSpec((1, cdim), lambda i: (0, 0))],
        out_specs=pl.BlockSpec((blk, cdim), lambda i: (i, 0)),
        out_shape=jax.ShapeDtypeStruct((n, cdim), jnp.float32),
    )(parts, d0, d1, bias2d)


def kernel(x, edge_index, W, bias):
    n, cdim = x.shape
    e = edge_index.shape[1]
    npad = ((n + 16 * NS - 1) // (16 * NS)) * (16 * NS)
    row1d = edge_index[0]
    col1d = edge_index[1]
    zeros = jnp.zeros((npad, cdim), jnp.float32)

    deg_flat = _hist(edge_index[1].reshape(NW, 1, e // NW), npad)
    d0 = deg_flat[:npad].reshape(npad, 1)
    d1 = deg_flat[npad:].reshape(npad, 1)
    xws = _matmul_scale(x, W, d0, d1)
    parts = _agg(xws, row1d, col1d, zeros)
    return _finalize(parts, d0, d1, bias.reshape(1, cdim), n)


# speedup vs baseline: 39.1335x; 1.0951x over previous
import functools

import jax
import jax.numpy as jnp
from jax import lax
from jax.experimental import pallas as pl
from jax.experimental.pallas import tpu as pltpu
from jax.experimental.pallas import tpu_sc as plsc

NC = 2
NS = 16
NW = NC * NS
CHUNK = 40
NBUF = 5


def _sc_mesh():
    return plsc.VectorSubcoreMesh(core_axis_name="c", subcore_axis_name="s")


def _hist(col2d, npad):
    _, _, per_w = col2d.shape
    rp = npad // NS

    @functools.partial(
        pl.kernel,
        out_type=jax.ShapeDtypeStruct((NC * npad,), jnp.float32),
        mesh=_sc_mesh(),
        compiler_params=pltpu.CompilerParams(needs_layout_passes=False),
        scratch_types=[
            pltpu.VMEM((1, per_w), jnp.int32),
            pltpu.VMEM((npad,), jnp.float32),
            pltpu.VMEM((rp,), jnp.float32),
            pltpu.VMEM_SHARED((NS * npad,), jnp.float32),
        ],
    )
    def hist_kernel(col_hbm, deg_hbm, idx_v, cnt_v, tmp_v, stage_sh):
        c = lax.axis_index("c")
        s = lax.axis_index("s")
        wid = s * NC + c
        pltpu.sync_copy(col_hbm.at[wid], idx_v)

        @pl.loop(0, npad // 16)
        def _(i):
            cnt_v[pl.ds(i * 16, 16)] = jnp.zeros((16,), jnp.float32)

        @pl.loop(0, per_w // 16)
        def _(k):
            idx = idx_v[0, pl.ds(k * 16, 16)]
            plsc.addupdate_scatter(cnt_v, [idx],
                                   jnp.full((16,), 1.0, jnp.float32))

        pltpu.sync_copy(cnt_v, stage_sh.at[pl.ds(s * npad, npad)])
        plsc.subcore_barrier()
        pltpu.sync_copy(stage_sh.at[pl.ds(s * rp, rp)], cnt_v.at[pl.ds(0, rp)])

        @pl.loop(1, NS)
        def _(t):
            pltpu.sync_copy(stage_sh.at[pl.ds(t * npad + s * rp, rp)], tmp_v)

            @pl.loop(0, rp // 16)
            def _(i):
                cnt_v[pl.ds(i * 16, 16)] = (cnt_v[pl.ds(i * 16, 16)]
                                            + tmp_v[pl.ds(i * 16, 16)])

        pltpu.sync_copy(cnt_v.at[pl.ds(0, rp)],
                        deg_hbm.at[pl.ds(c * npad + s * rp, rp)])

    return hist_kernel(col2d)


def _agg(xws, row1d, col1d, npad):
    cdim = xws.shape[1]
    chunk = CHUNK
    per_w = row1d.shape[0] // NW
    chunks_per_w = per_w // chunk
    rp = npad // NS
    assert rp % chunk == 0

    assert chunks_per_w % NBUF == 0
    nsteps = chunks_per_w // NBUF

    @functools.partial(
        pl.kernel,
        out_type=jax.ShapeDtypeStruct((NC, npad, cdim), jnp.float32),
        mesh=_sc_mesh(),
        scratch_types=[
            pltpu.VMEM((per_w,), jnp.int32),
            pltpu.VMEM((per_w,), jnp.int32),
        ] + [pltpu.VMEM((chunk, cdim), jnp.float32)] * NBUF
          + [pltpu.VMEM_SHARED((npad, cdim), jnp.float32)]
          + [pltpu.SemaphoreType.DMA] * (2 * NBUF),
    )
    def agg_kernel(xws_hbm, row_hbm, col_hbm, out_hbm,
                   row_v, col_v, *rest):
        bufs = rest[:NBUF]
        acc_sh = rest[NBUF]
        gsem = rest[NBUF + 1:2 * NBUF + 1]
        ssem = rest[2 * NBUF + 1:]
        c = lax.axis_index("c")
        s = lax.axis_index("s")
        wid = s * NC + c
        @pl.loop(0, chunk)
        def _(i):
            for k in range(cdim // 16):
                bufs[0][i, pl.ds(k * 16, 16)] = jnp.zeros((16,), jnp.float32)

        for k in range(rp // chunk):
            pltpu.async_copy(bufs[0],
                             acc_sh.at[pl.ds(s * rp + k * chunk, chunk)],
                             ssem[0])
        pltpu.sync_copy(row_hbm.at[pl.ds(wid * per_w, per_w)], row_v)
        pltpu.sync_copy(col_hbm.at[pl.ds(wid * per_w, per_w)], col_v)
        for k in range(rp // chunk):
            pltpu.make_async_copy(xws_hbm.at[col_v.at[pl.ds(0, chunk)]],
                                  bufs[0], ssem[0]).wait()

        def gather(j, b):
            pltpu.async_copy(xws_hbm.at[col_v.at[pl.ds(j * chunk, chunk)]],
                             bufs[b], gsem[b])

        def scatter(j, b):
            pltpu.async_copy(bufs[b],
                             acc_sh.at[row_v.at[pl.ds(j * chunk, chunk)]],
                             ssem[b], add=True)

        def wait(b, sem):
            pltpu.make_async_copy(xws_hbm.at[col_v.at[pl.ds(0, chunk)]],
                                  bufs[b], sem).wait()

        for b in range(NBUF):
            gather(b, b)
        plsc.subcore_barrier()

        @pl.loop(0, nsteps)
        def _(p):
            j0 = p * NBUF
            for b in range(NBUF):
                wait(b, gsem[b])
                scatter(j0 + b, b)
            for b in range(NBUF):
                wait(b, ssem[b])
                jn = j0 + NBUF + b

                @pl.when(jn < chunks_per_w)
                def _():
                    gather(jn, b)

        plsc.subcore_barrier()
        pltpu.sync_copy(acc_sh.at[pl.ds(s * rp, rp)],
                        out_hbm.at[c, pl.ds(s * rp, rp)])

    return agg_kernel(xws, row1d, col1d)


BLK = 1024


def _dis_rows(d_blk):
    deg = d_blk[0] + d_blk[1]
    return jnp.where(deg > 0.0, lax.rsqrt(deg), 0.0)


def _matmul_scale(x, w, d3):
    n, cin = x.shape
    cout = w.shape[0]
    npad = d3.shape[1] * 128

    def body(x_ref, w_ref, d_ref, o_ref):
        xw = lax.dot_general(
            x_ref[...], w_ref[...], (((1,), (1,)), ((), ())),
            preferred_element_type=jnp.float32)
        dis = _dis_rows(d_ref[...])
        xw3 = xw.reshape(BLK // 128, 128, cout)
        o_ref[...] = (xw3 * dis[:, :, None]).reshape(BLK, cout)

    return pl.pallas_call(
        body,
        grid=(npad // BLK,),
        in_specs=[pl.BlockSpec((BLK, cin), lambda i: (i, 0)),
                  pl.BlockSpec((cout, cin), lambda i: (0, 0)),
                  pl.BlockSpec((NC, BLK // 128, 128), lambda i: (0, i, 0))],
        out_specs=pl.BlockSpec((BLK, cout), lambda i: (i, 0)),
        out_shape=jax.ShapeDtypeStruct((n, cout), jnp.float32),
    )(x, w, d3)


def _finalize(parts, d3, bias2d, n):
    cdim = parts.shape[2]
    npad = parts.shape[1]

    def body(p_ref, d_ref, b_ref, o_ref):
        p = p_ref[0] + p_ref[1]
        dis = _dis_rows(d_ref[...])
        p3 = p.reshape(BLK // 128, 128, cdim)
        o_ref[...] = (p3 * dis[:, :, None]).reshape(BLK, cdim) + b_ref[...]

    return pl.pallas_call(
        body,
        grid=(npad // BLK,),
        in_specs=[pl.BlockSpec((NC, BLK, cdim), lambda i: (0, i, 0)),
                  pl.BlockSpec((NC, BLK // 128, 128), lambda i: (0, i, 0)),
                  pl.BlockSpec((1, cdim), lambda i: (0, 0))],
        out_specs=pl.BlockSpec((BLK, cdim), lambda i: (i, 0)),
        out_shape=jax.ShapeDtypeStruct((n, cdim), jnp.float32),
    )(parts, d3, bias2d)


def kernel(x, edge_index, W, bias):
    n, cdim = x.shape
    e = edge_index.shape[1]
    npad = ((n + 16 * NS - 1) // (16 * NS)) * (16 * NS)
    row1d = edge_index[0]
    col1d = edge_index[1]

    deg_flat = _hist(edge_index[1].reshape(NW, 1, e // NW), npad)
    d3 = deg_flat.reshape(NC, npad // 128, 128)
    xws = _matmul_scale(x, W, d3)
    parts = _agg(xws, row1d, col1d, npad)
    return _finalize(parts, d3, bias.reshape(1, cdim), n)


# speedup vs baseline: 39.4024x; 1.0069x over previous
import functools

import jax
import jax.numpy as jnp
from jax import lax
from jax.experimental import pallas as pl
from jax.experimental.pallas import tpu as pltpu
from jax.experimental.pallas import tpu_sc as plsc

NC = 2
NS = 16
NW = NC * NS
CHUNK = 40
NBUF = 5


def _sc_mesh():
    return plsc.VectorSubcoreMesh(core_axis_name="c", subcore_axis_name="s")


def _hist(col2d, npad):
    _, _, per_w = col2d.shape
    rp = npad // NS

    @functools.partial(
        pl.kernel,
        out_type=jax.ShapeDtypeStruct((NC * npad,), jnp.float32),
        mesh=_sc_mesh(),
        compiler_params=pltpu.CompilerParams(needs_layout_passes=False),
        scratch_types=[
            pltpu.VMEM((1, per_w), jnp.int32),
            pltpu.VMEM((npad,), jnp.float32),
            pltpu.VMEM((rp,), jnp.float32),
            pltpu.VMEM((rp,), jnp.float32),
            pltpu.VMEM_SHARED((NS * npad,), jnp.float32),
            pltpu.SemaphoreType.DMA,
            pltpu.SemaphoreType.DMA,
        ],
    )
    def hist_kernel(col_hbm, deg_hbm, idx_v, cnt_v, tmp_a, tmp_b, stage_sh,
                    sem_a, sem_b):
        c = lax.axis_index("c")
        s = lax.axis_index("s")
        wid = s * NC + c
        pltpu.sync_copy(col_hbm.at[wid], idx_v)

        @pl.loop(0, npad // 16)
        def _(i):
            cnt_v[pl.ds(i * 16, 16)] = jnp.zeros((16,), jnp.float32)

        @pl.loop(0, per_w // 16)
        def _(k):
            idx = idx_v[0, pl.ds(k * 16, 16)]
            plsc.addupdate_scatter(cnt_v, [idx],
                                   jnp.full((16,), 1.0, jnp.float32))

        pltpu.sync_copy(cnt_v, stage_sh.at[pl.ds(s * npad, npad)])
        plsc.subcore_barrier()
        tmps = (tmp_a, tmp_b)
        sems = (sem_a, sem_b)

        def load(t, b):
            pltpu.async_copy(stage_sh.at[pl.ds(t * npad + s * rp, rp)],
                             tmps[b], sems[b])

        load(0, 0)
        load(1, 1)
        for t in range(NS):
            b = t % 2
            pltpu.make_async_copy(stage_sh.at[pl.ds(s * rp, rp)],
                                  tmps[b], sems[b]).wait()
            if t == 0:
                @pl.loop(0, rp // 16)
                def _(i):
                    cnt_v[pl.ds(i * 16, 16)] = tmps[0][pl.ds(i * 16, 16)]
            else:
                tb = tmps[b]

                @pl.loop(0, rp // 16)
                def _(i):
                    cnt_v[pl.ds(i * 16, 16)] = (cnt_v[pl.ds(i * 16, 16)]
                                                + tb[pl.ds(i * 16, 16)])
            if t + 2 < NS:
                load(t + 2, b)

        pltpu.sync_copy(cnt_v.at[pl.ds(0, rp)],
                        deg_hbm.at[pl.ds(c * npad + s * rp, rp)])

    return hist_kernel(col2d)


def _agg(xws, row1d, col1d, npad):
    cdim = xws.shape[1]
    chunk = CHUNK
    per_w = row1d.shape[0] // NW
    chunks_per_w = per_w // chunk
    rp = npad // NS
    assert rp % chunk == 0

    assert chunks_per_w % NBUF == 0
    nsteps = chunks_per_w // NBUF

    @functools.partial(
        pl.kernel,
        out_type=jax.ShapeDtypeStruct((NC, npad, cdim), jnp.float32),
        mesh=_sc_mesh(),
        scratch_types=[
            pltpu.VMEM((per_w,), jnp.int32),
            pltpu.VMEM((per_w,), jnp.int32),
        ] + [pltpu.VMEM((chunk, cdim), jnp.float32)] * NBUF
          + [pltpu.VMEM_SHARED((npad, cdim), jnp.float32)]
          + [pltpu.SemaphoreType.DMA] * (2 * NBUF),
    )
    def agg_kernel(xws_hbm, row_hbm, col_hbm, out_hbm,
                   row_v, col_v, *rest):
        bufs = rest[:NBUF]
        acc_sh = rest[NBUF]
        gsem = rest[NBUF + 1:2 * NBUF + 1]
        ssem = rest[2 * NBUF + 1:]
        c = lax.axis_index("c")
        s = lax.axis_index("s")
        wid = s * NC + c
        @pl.loop(0, chunk)
        def _(i):
            for k in range(cdim // 16):
                bufs[0][i, pl.ds(k * 16, 16)] = jnp.zeros((16,), jnp.float32)

        for k in range(rp // chunk):
            pltpu.async_copy(bufs[0],
                             acc_sh.at[pl.ds(s * rp + k * chunk, chunk)],
                             ssem[0])
        pltpu.sync_copy(row_hbm.at[pl.ds(wid * per_w, per_w)], row_v)
        pltpu.sync_copy(col_hbm.at[pl.ds(wid * per_w, per_w)], col_v)
        for k in range(rp // chunk):
            pltpu.make_async_copy(xws_hbm.at[col_v.at[pl.ds(0, chunk)]],
                                  bufs[0], ssem[0]).wait()

        def gather(j, b):
            pltpu.async_copy(xws_hbm.at[col_v.at[pl.ds(j * chunk, chunk)]],
                             bufs[b], gsem[b])

        def scatter(j, b):
            pltpu.async_copy(bufs[b],
                             acc_sh.at[row_v.at[pl.ds(j * chunk, chunk)]],
                             ssem[b], add=True)

        def wait(b, sem):
            pltpu.make_async_copy(xws_hbm.at[col_v.at[pl.ds(0, chunk)]],
                                  bufs[b], sem).wait()

        for b in range(NBUF):
            gather(b, b)
        plsc.subcore_barrier()

        @pl.loop(0, nsteps)
        def _(p):
            j0 = p * NBUF
            for b in range(NBUF):
                wait(b, gsem[b])
                scatter(j0 + b, b)
            for b in range(NBUF):
                wait(b, ssem[b])
                jn = j0 + NBUF + b

                @pl.when(jn < chunks_per_w)
                def _():
                    gather(jn, b)

        plsc.subcore_barrier()
        pltpu.sync_copy(acc_sh.at[pl.ds(s * rp, rp)],
                        out_hbm.at[c, pl.ds(s * rp, rp)])

    return agg_kernel(xws, row1d, col1d)


BLK = 1024


def _dis_rows(d_blk):
    deg = d_blk[0] + d_blk[1]
    return jnp.where(deg > 0.0, lax.rsqrt(deg), 0.0)


def _matmul_scale(x, w, d3):
    n, cin = x.shape
    cout = w.shape[0]
    npad = d3.shape[1] * 128

    def body(x_ref, w_ref, d_ref, o_ref):
        xw = lax.dot_general(
            x_ref[...], w_ref[...], (((1,), (1,)), ((), ())),
            preferred_element_type=jnp.float32)
        dis = _dis_rows(d_ref[...])
        xw3 = xw.reshape(BLK // 128, 128, cout)
        o_ref[...] = (xw3 * dis[:, :, None]).reshape(BLK, cout)

    return pl.pallas_call(
        body,
        grid=(npad // BLK,),
        in_specs=[pl.BlockSpec((BLK, cin), lambda i: (i, 0)),
                  pl.BlockSpec((cout, cin), lambda i: (0, 0)),
                  pl.BlockSpec((NC, BLK // 128, 128), lambda i: (0, i, 0))],
        out_specs=pl.BlockSpec((BLK, cout), lambda i: (i, 0)),
        out_shape=jax.ShapeDtypeStruct((n, cout), jnp.float32),
    )(x, w, d3)


def _finalize(parts, d3, bias2d, n):
    cdim = parts.shape[2]
    npad = parts.shape[1]

    def body(p_ref, d_ref, b_ref, o_ref):
        p = p_ref[0] + p_ref[1]
        dis = _dis_rows(d_ref[...])
        p3 = p.reshape(BLK // 128, 128, cdim)
        o_ref[...] = (p3 * dis[:, :, None]).reshape(BLK, cdim) + b_ref[...]

    return pl.pallas_call(
        body,
        grid=(npad // BLK,),
        in_specs=[pl.BlockSpec((NC, BLK, cdim), lambda i: (0, i, 0)),
                  pl.BlockSpec((NC, BLK // 128, 128), lambda i: (0, i, 0)),
                  pl.BlockSpec((1, cdim), lambda i: (0, 0))],
        out_specs=pl.BlockSpec((BLK, cdim), lambda i: (i, 0)),
        out_shape=jax.ShapeDtypeStruct((n, cdim), jnp.float32),
    )(parts, d3, bias2d)


def kernel(x, edge_index, W, bias):
    n, cdim = x.shape
    e = edge_index.shape[1]
    npad = ((n + 16 * NS - 1) // (16 * NS)) * (16 * NS)
    row1d = edge_index[0]
    col1d = edge_index[1]

    deg_flat = _hist(edge_index[1].reshape(NW, 1, e // NW), npad)
    d3 = deg_flat.reshape(NC, npad // 128, 128)
    xws = _matmul_scale(x, W, d3)
    parts = _agg(xws, row1d, col1d, npad)
    return _finalize(parts, d3, bias.reshape(1, cdim), n)
